# baseline (speedup 1.0000x reference)
import functools

import jax
import jax.numpy as jnp
from jax import lax
from jax.experimental import pallas as pl
from jax.experimental.pallas import tpu as pltpu
from jax.experimental.pallas import tpu_sc as plsc

B = 16
C = 80
NT = 50
NTP = 64
SCALES = [
    (80, 8.0, 0.0, 64.0),
    (40, 16.0, 64.0, 128.0),
    (20, 32.0, 128.0, 10000.0),
]
NTASK = 3 * B * 4
ROWS_CLS = C * 16
ROWS_ALL = ROWS_CLS + 4 * 16 + 16
S0_ALL = 112
HP = [80, 40, 24]


def _relayout(c3, b3, o3, c4, b4, o4, c5, b5, o5):
    def body(c3r, b3r, o3r, c4r, b4r, o4r, c5r, b5r, o5r,
             c3o, b3o, o3o, c4o, b4o, o4o, c5o, b5o, o5o,
             s3o, s4o, s5o, f3o):
        for (ir, orf) in ((b3r, b3o), (o3r, o3o), (c4r, c4o),
                          (b4r, b4o), (o4r, o4o), (b5r, b5o), (o5r, o5o)):
            x = ir[...]
            pads = tuple((0, do - di) for di, do in zip(x.shape, orf.shape))
            orf[...] = jnp.pad(x, pads)
        x3 = c3r[...]
        fsum = jnp.sum(_softplus(x3), axis=1)
        f3o[...] = jnp.pad(fsum, ((0, 0), (0, 0), (0, 48)))
        y16 = lax.bitcast_convert_type(x3.astype(jnp.bfloat16), jnp.uint16)
        y2 = y16.reshape(1, C // 2, 2, 80, 80)
        w32 = (y2[:, :, 0].astype(jnp.uint32)
               | (y2[:, :, 1].astype(jnp.uint32) << 16))
        c3o[...] = jnp.pad(lax.bitcast_convert_type(w32, jnp.int32),
                           ((0, 0), (0, 0), (0, 0), (0, 48)))

        @pl.when(pl.program_id(0) == 0)
        def _():
            x = c5r[...]
            pads = tuple((0, do - di) for di, do in zip(x.shape, c5o.shape))
            c5o[...] = jnp.pad(x, pads)

        for orf, sref in ((o3r, s3o), (o4r, s4o), (o5r, s5o)):
            sp = _softplus(orf[...][0, 0])
            r = jnp.sum(sp, axis=1, keepdims=True)
            sref[...] = jnp.broadcast_to(
                jnp.sum(r, axis=0, keepdims=True)[None], (1, 8, 128))

    ins = (c3, b3, o3, c4, b4, o4, c5, b5, o5)
    out_dims = [(1, C // 2, 80, 128), (1, 4, 80, 128), (1, 1, 80, 128),
                (1, 40, 40, 128), (1, 4, 40, 128), (1, 1, 40, 128),
                (20, 20, B, 128), (1, 4, 24, 128), (1, 1, 24, 128)]
    bdim = [0, 0, 0, 0, 0, 0, None, 0, 0]
    in_specs = []
    out_shapes = []
    out_specs = []
    for a, od, bd in zip(ins, out_dims, bdim):
        if bd is None:
            in_specs.append(pl.BlockSpec(a.shape, lambda b: (0, 0, 0, 0)))
            out_shapes.append(jax.ShapeDtypeStruct(od, jnp.float32))
            out_specs.append(pl.BlockSpec(od, lambda b: (0, 0, 0, 0)))
            continue
        blk_in = tuple(1 if i == bd else d for i, d in enumerate(a.shape))
        imap = lambda b: (b, 0, 0, 0)
        in_specs.append(pl.BlockSpec(blk_in, imap))
        full = tuple(B if i == bd else d for i, d in enumerate(od))
        out_shapes.append(jax.ShapeDtypeStruct(full, jnp.float32))
        out_specs.append(pl.BlockSpec(od, imap))
    out_shapes[0] = jax.ShapeDtypeStruct((B, C // 2, 80, 128), jnp.int32)
    for _ in range(3):
        out_shapes.append(jax.ShapeDtypeStruct((B, 8, 128), jnp.float32))
        out_specs.append(pl.BlockSpec((1, 8, 128), lambda b: (b, 0, 0)))
    out_shapes.append(jax.ShapeDtypeStruct((B, 80, 128), jnp.float32))
    out_specs.append(pl.BlockSpec((1, 80, 128), lambda b: (b, 0, 0)))
    (c3o, b3o, o3o, c4o, b4o, o4o, c5o, b5o, o5o, s3o, s4o, s5o,
     f3o) = pl.pallas_call(
        body, grid=(B,), in_specs=in_specs, out_specs=out_specs,
        out_shape=out_shapes)(*ins)
    return ((f3o, c3o, c4o, c5o, b3o, b4o, b5o, o3o, o4o, o5o),
            (s3o, s4o, s5o))


def _sc_gather(sumf3, pk3, cls4, cls5, box3, box4, box5, obj3, obj4, obj5,
               boxes_sc):
    mesh = plsc.VectorSubcoreMesh(core_axis_name="c", subcore_axis_name="s")

    @functools.partial(
        pl.kernel,
        mesh=mesh,
        out_type=jax.ShapeDtypeStruct((3, 4, B, ROWS_ALL), jnp.float32),
        scratch_types=[
            pltpu.VMEM((2, 5, 16), jnp.float32),
            pltpu.VMEM((2, 10, 128), jnp.int32),
            pltpu.VMEM((2, 1, 64), jnp.int32),
            pltpu.VMEM((2, 1, 16), jnp.int32),
            pltpu.VMEM((2, 1, 16), jnp.int32),
            pltpu.VMEM((2, 1, 16), jnp.int32),
            pltpu.VMEM((2, ROWS_ALL), jnp.float32),
            pltpu.SemaphoreType.DMA,
            pltpu.SemaphoreType.DMA,
        ],
    )
    def k(sumf3_r, pk3_r, cls4_r, cls5_r, box3_r, box4_r, box5_r,
          obj3_r, obj4_r, obj5_r, bsc_r, out_r,
          bx_v, idxc_v, idxb_v, idxo_v, idxf_v, idxp_v, vals_v, sem0, sem1):
        wid = lax.axis_index("s") * 2 + lax.axis_index("c")
        sems = [sem0, sem1]
        tabs = [(None, box3_r, obj3_r),
                (cls4_r, box4_r, obj4_r),
                (cls5_r, box5_r, obj5_r)]
        for rep in range(2):
            pltpu.sync_copy(bsc_r.at[wid + 32 * rep], bx_v.at[rep])
        tasks = [(s, rep) for s in range(3) for rep in range(2)]
        handles = [None] * 6

        def writeout(i):
            s_i, rep_i = tasks[i]
            slot_i = i % 2
            u_i = wid + 32 * rep_i
            n = S0_ALL if s_i == 0 else ROWS_ALL
            pltpu.sync_copy(
                vals_v.at[slot_i, pl.ds(0, n)],
                out_r.at[s_i, u_i % 4, u_i // 4, pl.ds(0, n)])

        for i, (s, rep) in enumerate(tasks):
            slot = i % 2
            if i >= 2:
                for cp in handles[i - 2]:
                    cp.wait()
                writeout(i - 2)
            w_dim = SCALES[s][0]
            hw = HP[s] * 128
            cls_tab, box_tab, obj_tab = tabs[s]
            u = wid + 32 * rep
            b = u // 4
            cx = bx_v[rep, 0, :]
            cy = bx_v[rep, 1, :]
            gx = jnp.clip((cx * float(w_dim)).astype(jnp.int32), 0, w_dim - 1)
            gy = jnp.clip((cy * float(w_dim)).astype(jnp.int32), 0, w_dim - 1)
            pos = gy * 128 + gx
            hs = []
            if s == 0:
                lab = bx_v[rep, 4, :].astype(jnp.int32)
                c2 = lax.shift_right_logical(lab, 1)
                idxf_v[slot, 0, :] = b * (80 * 128) + pos
                idxp_v[slot, 0, :] = (b * 40 + c2) * (80 * 128) + pos
                box_off, obj_off = 32, 96
            else:
                if s == 1:
                    base_cls = b * (40 * 40 * 128) + (gy * 40 + gx) * 128
                else:
                    base_cls = (gy * 20 + gx) * (B * 128) + b * 128

                def build_cls(kk, carry, base_cls=base_cls, slot=slot):
                    for j in range(8):
                        idxc_v[slot, kk, pl.ds(j * 16, 16)] = (
                            base_cls + (kk * 8 + j))
                    return carry
                lax.fori_loop(0, 10, build_cls, 0)
                box_off, obj_off = ROWS_CLS, ROWS_CLS + 64
            for c4 in range(4):
                idxb_v[slot, 0, pl.ds(c4 * 16, 16)] = (
                    b * (4 * hw) + c4 * hw + pos)
            idxo_v[slot, 0, :] = b * hw + pos
            if s == 0:
                hs.append(pltpu.async_copy(
                    sumf3_r.at[idxf_v.at[slot, 0]],
                    vals_v.at[slot, pl.ds(0, 16)], sems[slot]))
                hs.append(pltpu.async_copy(
                    pk3_r.at[idxp_v.at[slot, 0]],
                    vals_v.at[slot, pl.ds(16, 16)], sems[slot]))
            else:
                for kk in range(10):
                    hs.append(pltpu.async_copy(
                        cls_tab.at[idxc_v.at[slot, kk]],
                        vals_v.at[slot, pl.ds(kk * 128, 128)], sems[slot]))
            hs.append(pltpu.async_copy(
                box_tab.at[idxb_v.at[slot, 0]],
                vals_v.at[slot, pl.ds(box_off, 64)], sems[slot]))
            hs.append(pltpu.async_copy(
                obj_tab.at[idxo_v.at[slot, 0]],
                vals_v.at[slot, pl.ds(obj_off, 16)], sems[slot]))
            handles[i] = hs
        for i in (4, 5):
            for cp in handles[i]:
                cp.wait()
            writeout(i)

    return k(sumf3, pk3, cls4, cls5, box3, box4, box5, obj3, obj4, obj5,
             boxes_sc)


def _sigmoid(x):
    return 1.0 / (1.0 + jnp.exp(-x))


def _l1pe(x):
    return jnp.log(1.0 + jnp.exp(-jnp.abs(x)))


def _softplus(x):
    return jnp.maximum(x, 0.0) + _l1pe(x)


def _atan(x):
    w1 = x / (1.0 + jnp.sqrt(1.0 + x * x))
    w = w1 / (1.0 + jnp.sqrt(1.0 + w1 * w1))
    w2 = w * w
    p = w * (1.0 + w2 * (-1.0 / 3.0 + w2 * (1.0 / 5.0 + w2 * (
        -1.0 / 7.0 + w2 * (1.0 / 9.0 + w2 * (-1.0 / 11.0 + w2 / 13.0))))))
    return 4.0 * p


def _tc_body(g_ref, o3_ref, o4_ref, o5_ref, btc_ref, lab_ref, img_ref,
             lo_ref, lbx_ref, lob_ref, lcl_ref):
    img = img_ref[0, 0]
    eps = 1e-7
    seg = (lax.broadcasted_iota(jnp.int32, (ROWS_CLS, 16), 0) % 16
           == lax.broadcasted_iota(jnp.int32, (ROWS_CLS, 16), 1)
           ).astype(jnp.float32)
    total_box = jnp.float32(0.0)
    total_obj = jnp.float32(0.0)
    total_cls = jnp.float32(0.0)
    total_pos = jnp.float32(0.0)
    o_refs = (o3_ref, o4_ref, o5_ref)
    for s, (w_dim, stride, mn, mx) in enumerate(SCALES):
        hw = w_dim * w_dim
        obj_sum = o_refs[s][:, 0, 0:1]
        box_b = jnp.zeros((B, 1), jnp.float32)
        cls_b = jnp.zeros((B, 1), jnp.float32)
        pos_b = jnp.zeros((B, 1), jnp.float32)
        nval_b = jnp.zeros((B, 1), jnp.float32)
        pos_l, mf_l, spo_l = [], [], []
        for ck in range(4):
            sl = slice(ck * 16, (ck + 1) * 16)
            cx = btc_ref[0, :, sl]
            cy = btc_ref[1, :, sl]
            tw = btc_ref[2, :, sl]
            th = btc_ref[3, :, sl]
            tmax = jnp.maximum(tw, th) * img
            maskf = ((tmax > mn) & (tmax <= mx)).astype(jnp.float32)
            tx1 = cx - 0.5 * tw
            ty1 = cy - 0.5 * th
            tx2 = cx + 0.5 * tw
            ty2 = cy + 0.5 * th
            gx = jnp.clip((cx * w_dim).astype(jnp.int32), 0, w_dim - 1)
            gy = jnp.clip((cy * w_dim).astype(jnp.int32), 0, w_dim - 1)
            yk = g_ref[s, ck]
            box_off, obj_off = (32, 96) if s == 0 else (ROWS_CLS,
                                                       ROWS_CLS + 64)
            r0 = yk[:, box_off:box_off + 16]
            r1 = yk[:, box_off + 16:box_off + 32]
            r2 = yk[:, box_off + 32:box_off + 48]
            r3 = yk[:, box_off + 48:box_off + 64]
            gobj = yk[:, obj_off:obj_off + 16]
            px = (gx.astype(jnp.float32) + _sigmoid(r0)) * stride / img
            py = (gy.astype(jnp.float32) + _sigmoid(r1)) * stride / img
            pw = jnp.exp(r2) * stride / img
            ph = jnp.exp(r3) * stride / img
            px1 = px - 0.5 * pw
            py1 = py - 0.5 * ph
            px2 = px + 0.5 * pw
            py2 = py + 0.5 * ph
            ap = (px2 - px1) * (py2 - py1)
            at = (tx2 - tx1) * (ty2 - ty1)
            iw = jnp.clip(jnp.minimum(px2, tx2) - jnp.maximum(px1, tx1),
                          0.0, None)
            ih = jnp.clip(jnp.minimum(py2, ty2) - jnp.maximum(py1, ty1),
                          0.0, None)
            inter = iw * ih
            union = ap + at - inter + eps
            iou = inter / union
            cw = jnp.maximum(px2, tx2) - jnp.minimum(px1, tx1)
            ch = jnp.maximum(py2, ty2) - jnp.minimum(py1, ty1)
            c2 = cw * cw + ch * ch + eps
            rho2 = ((px1 + px2 - tx1 - tx2) ** 2
                    + (py1 + py2 - ty1 - ty2) ** 2) / 4.0
            v = (4.0 / (jnp.pi ** 2)) * (
                _atan((tx2 - tx1) / (ty2 - ty1 + eps))
                - _atan((px2 - px1) / (py2 - py1 + eps))) ** 2
            alpha = v / (1.0 - iou + v + eps)
            ciou = iou - rho2 / c2 - alpha * v
            box_b = box_b + jnp.sum((1.0 - jnp.clip(ciou, -1.0, 1.0)) * maskf,
                                    axis=1, keepdims=True)
            if s == 0:
                sumf_t = yk[:, 0:16]
                packed = lax.bitcast_convert_type(yk[:, 16:32], jnp.uint32)
                labs = lab_ref[ck][:, 0:16]
                sel = jnp.where((labs & 1) == 1, packed >> 16,
                                packed & 0xFFFF)
                xlab = lax.bitcast_convert_type(
                    sel.astype(jnp.uint16), jnp.bfloat16).astype(jnp.float32)
                cls_t = sumf_t - xlab
            else:
                gcls = yk[:, 0:ROWS_CLS]
                labx = lab_ref[ck]
                cj = lax.broadcasted_iota(jnp.int32, (B, ROWS_CLS), 1) // 16
                oneh = (cj == labx).astype(jnp.float32)
                bce_cls = jnp.maximum(gcls, 0.0) - gcls * oneh + _l1pe(gcls)
                cls_t = jax.lax.dot(bce_cls, seg,
                                    precision=jax.lax.Precision.HIGHEST)
            cls_b = cls_b + jnp.sum(cls_t * maskf, axis=1, keepdims=True)
            pos_b = pos_b + jnp.sum(
                (jnp.maximum(gobj, 0.0) - gobj + _l1pe(gobj)) * maskf,
                axis=1, keepdims=True)
            nval_b = nval_b + jnp.sum(maskf, axis=1, keepdims=True)
            pos_l.append(gy * w_dim + gx)
            mf_l.append(maskf)
            spo_l.append(_softplus(gobj))
        neg_hit = jnp.zeros((B, 1), jnp.float32)
        for ck in range(4):
            dup = jnp.zeros((B, 16, 16), jnp.float32)
            for pk in range(ck + 1):
                eq = pos_l[pk][:, :, None] == pos_l[ck][:, None, :]
                if pk == ck:
                    tri = (lax.broadcasted_iota(jnp.int32, (B, 16, 16), 1)
                           < lax.broadcasted_iota(jnp.int32, (B, 16, 16), 2))
                    eq = eq & tri
                dup = dup + jnp.where(eq, mf_l[pk][:, :, None], 0.0)
            first = mf_l[ck] * (jnp.sum(dup, axis=1) == 0.0).astype(jnp.float32)
            neg_hit = neg_hit + jnp.sum(spo_l[ck] * first,
                                        axis=1, keepdims=True)
        neg_b = 0.05 * (obj_sum - neg_hit)
        fb_b = 0.1 * obj_sum / hw
        has = nval_b > 0.0
        total_obj = total_obj + jnp.sum(jnp.where(has, pos_b + neg_b, fb_b))
        total_box = total_box + jnp.sum(jnp.where(has, box_b, 0.0))
        total_cls = total_cls + jnp.sum(jnp.where(has, cls_b, 0.0))
        total_pos = total_pos + jnp.sum(jnp.where(has, nval_b, 0.0))
    norm = jnp.maximum(total_pos, 1.0)
    box_loss = total_box / norm
    cls_loss = total_cls / norm
    obj_loss = total_obj / (B * 3.0)
    loss = 7.5 * box_loss + 1.0 * obj_loss + 0.5 * cls_loss
    lo_ref[...] = jnp.reshape(loss, (1, 1))
    lbx_ref[...] = jnp.reshape(box_loss, (1, 1))
    lob_ref[...] = jnp.reshape(obj_loss, (1, 1))
    lcl_ref[...] = jnp.reshape(cls_loss, (1, 1))


def _tc_loss(g, o3, o4, o5, btc, lab_exp, img11):
    return pl.pallas_call(
        _tc_body,
        out_shape=[jax.ShapeDtypeStruct((1, 1), jnp.float32)] * 4,
    )(g, o3, o4, o5, btc, lab_exp, img11)


def kernel(p3_bbox, p3_obj, p3_cls, p4_bbox, p4_obj, p4_cls,
           p5_bbox, p5_obj, p5_cls, boxes, labels, img_size):
    img = jnp.asarray(img_size).astype(jnp.float32)
    boxes_p = jnp.pad(boxes.astype(jnp.float32),
                      ((0, 0), (0, NTP - NT), (0, 0)))
    labels_p = jnp.pad(labels.astype(jnp.int32), ((0, 0), (0, NTP - NT)))
    boxes_sc = jnp.concatenate([
        boxes_p.reshape(B, 4, 16, 4).transpose(0, 1, 3, 2).reshape(64, 4, 16),
        labels_p.astype(jnp.float32).reshape(64, 1, 16)], axis=1)
    boxes_tc = boxes_p.transpose(2, 0, 1)
    lin, osums = _relayout(p3_cls, p3_bbox, p3_obj,
                           p4_cls.transpose(0, 2, 3, 1), p4_bbox, p4_obj,
                           p5_cls.transpose(2, 3, 0, 1), p5_bbox, p5_obj)
    lin = (lin[0], lax.bitcast_convert_type(lin[1], jnp.float32)) + lin[2:]
    g = _sc_gather(*[t.reshape(-1) for t in lin], boxes_sc)
    lab_exp = jnp.broadcast_to(
        labels_p.reshape(B, 4, 1, 16).transpose(1, 0, 2, 3),
        (4, B, C, 16)).reshape(4, B, ROWS_CLS)
    loss, box_loss, obj_loss, cls_loss = _tc_loss(
        g, *osums, boxes_tc, lab_exp, img.reshape(1, 1))
    return (loss.reshape(()), box_loss.reshape(()),
            obj_loss.reshape(()), cls_loss.reshape(()))


# speedup vs baseline: 1.1747x; 1.1747x over previous
import functools

import jax
import jax.numpy as jnp
from jax import lax
from jax.experimental import pallas as pl
from jax.experimental.pallas import tpu as pltpu
from jax.experimental.pallas import tpu_sc as plsc

B = 16
C = 80
NT = 50
NTP = 64
SCALES = [
    (80, 8.0, 0.0, 64.0),
    (40, 16.0, 64.0, 128.0),
    (20, 32.0, 128.0, 10000.0),
]
NTASK = 3 * B * 4
ROWS_CLS = C * 16
ROWS_ALL = ROWS_CLS + 4 * 16 + 16
S0_ALL = 112
HP = [80, 40, 24]


def _relayout(c3, b3, o3, c4, b4, o4, c5, b5, o5):
    def body(c3r, b3r, o3r, c4r, b4r, o4r, c5r, b5r, o5r,
             c3o, b3o, o3o, c4o, b4o, o4o, c5o, b5o, o5o,
             s3o, s4o, s5o, f3o):
        for (ir, orf) in ((b3r, b3o), (o3r, o3o), (c4r, c4o),
                          (b4r, b4o), (o4r, o4o), (b5r, b5o), (o5r, o5o)):
            x = ir[...]
            pads = tuple((0, do - di) for di, do in zip(x.shape, orf.shape))
            orf[...] = jnp.pad(x, pads)
        x3 = c3r[...]
        fsum = jnp.sum(_softplus(x3), axis=1)
        f3o[...] = jnp.pad(fsum, ((0, 0), (0, 0), (0, 48)))
        y16 = lax.bitcast_convert_type(x3.astype(jnp.bfloat16), jnp.uint16)
        y2 = y16.reshape(1, C // 2, 2, 80, 80)
        w32 = (y2[:, :, 0].astype(jnp.uint32)
               | (y2[:, :, 1].astype(jnp.uint32) << 16))
        c3o[...] = jnp.pad(lax.bitcast_convert_type(w32, jnp.float32),
                           ((0, 0), (0, 0), (0, 0), (0, 48)))

        @pl.when(pl.program_id(0) == 0)
        def _():
            x = c5r[...]
            pads = tuple((0, do - di) for di, do in zip(x.shape, c5o.shape))
            c5o[...] = jnp.pad(x, pads)

        for orf, sref in ((o3r, s3o), (o4r, s4o), (o5r, s5o)):
            sp = _softplus(orf[...][0, 0])
            r = jnp.sum(sp, axis=1, keepdims=True)
            sref[...] = jnp.broadcast_to(
                jnp.sum(r, axis=0, keepdims=True)[None], (1, 8, 128))

    ins = (c3, b3, o3, c4, b4, o4, c5, b5, o5)
    out_dims = [(1, C // 2, 80, 128), (1, 4, 80, 128), (1, 1, 80, 128),
                (1, 40, 40, 128), (1, 4, 40, 128), (1, 1, 40, 128),
                (20, 20, B, 128), (1, 4, 24, 128), (1, 1, 24, 128)]
    bdim = [0, 0, 0, 0, 0, 0, None, 0, 0]
    in_specs = []
    out_shapes = []
    out_specs = []
    for a, od, bd in zip(ins, out_dims, bdim):
        if bd is None:
            in_specs.append(pl.BlockSpec(a.shape, lambda b: (0, 0, 0, 0)))
            out_shapes.append(jax.ShapeDtypeStruct(od, jnp.float32))
            out_specs.append(pl.BlockSpec(od, lambda b: (0, 0, 0, 0)))
            continue
        blk_in = tuple(1 if i == bd else d for i, d in enumerate(a.shape))
        imap = lambda b: (b, 0, 0, 0)
        in_specs.append(pl.BlockSpec(blk_in, imap))
        full = tuple(B if i == bd else d for i, d in enumerate(od))
        out_shapes.append(jax.ShapeDtypeStruct(full, jnp.float32))
        out_specs.append(pl.BlockSpec(od, imap))
    pass
    for _ in range(3):
        out_shapes.append(jax.ShapeDtypeStruct((B, 8, 128), jnp.float32))
        out_specs.append(pl.BlockSpec((1, 8, 128), lambda b: (b, 0, 0)))
    out_shapes.append(jax.ShapeDtypeStruct((B, 80, 128), jnp.float32))
    out_specs.append(pl.BlockSpec((1, 80, 128), lambda b: (b, 0, 0)))
    (c3o, b3o, o3o, c4o, b4o, o4o, c5o, b5o, o5o, s3o, s4o, s5o,
     f3o) = pl.pallas_call(
        body, grid=(B,), in_specs=in_specs, out_specs=out_specs,
        out_shape=out_shapes)(*ins)
    return ((f3o, c3o, c4o, c5o, b3o, b4o, b5o, o3o, o4o, o5o),
            (s3o, s4o, s5o))


def _sc_gather(sumf3, pk3, cls4, cls5, box3, box4, box5, obj3, obj4, obj5,
               boxes_sc):
    mesh = plsc.VectorSubcoreMesh(core_axis_name="c", subcore_axis_name="s")

    @functools.partial(
        pl.kernel,
        mesh=mesh,
        out_type=jax.ShapeDtypeStruct((3, 4, B, ROWS_ALL), jnp.float32),
        scratch_types=[
            pltpu.VMEM((2, 5, 16), jnp.float32),
            pltpu.VMEM((2, 10, 128), jnp.int32),
            pltpu.VMEM((2, 1, 64), jnp.int32),
            pltpu.VMEM((2, 1, 16), jnp.int32),
            pltpu.VMEM((2, 1, 16), jnp.int32),
            pltpu.VMEM((2, 1, 16), jnp.int32),
            pltpu.VMEM((2, ROWS_ALL), jnp.float32),
            pltpu.SemaphoreType.DMA,
            pltpu.SemaphoreType.DMA,
        ],
    )
    def k(sumf3_r, pk3_r, cls4_r, cls5_r, box3_r, box4_r, box5_r,
          obj3_r, obj4_r, obj5_r, bsc_r, out_r,
          bx_v, idxc_v, idxb_v, idxo_v, idxf_v, idxp_v, vals_v, sem0, sem1):
        wid = lax.axis_index("s") * 2 + lax.axis_index("c")
        sems = [sem0, sem1]
        tabs = [(None, box3_r, obj3_r),
                (cls4_r, box4_r, obj4_r),
                (cls5_r, box5_r, obj5_r)]
        for rep in range(2):
            pltpu.sync_copy(bsc_r.at[wid + 32 * rep], bx_v.at[rep])
        tasks = [(s, rep) for s in range(3) for rep in range(2)]
        handles = [None] * 6

        def writeout(i):
            s_i, rep_i = tasks[i]
            slot_i = i % 2
            u_i = wid + 32 * rep_i
            n = S0_ALL if s_i == 0 else ROWS_ALL
            pltpu.sync_copy(
                vals_v.at[slot_i, pl.ds(0, n)],
                out_r.at[s_i, u_i % 4, u_i // 4, pl.ds(0, n)])

        for i, (s, rep) in enumerate(tasks):
            slot = i % 2
            if i >= 2:
                for cp in handles[i - 2]:
                    cp.wait()
                writeout(i - 2)
            w_dim = SCALES[s][0]
            hw = HP[s] * 128
            cls_tab, box_tab, obj_tab = tabs[s]
            u = wid + 32 * rep
            b = u // 4
            cx = bx_v[rep, 0, :]
            cy = bx_v[rep, 1, :]
            gx = jnp.clip((cx * float(w_dim)).astype(jnp.int32), 0, w_dim - 1)
            gy = jnp.clip((cy * float(w_dim)).astype(jnp.int32), 0, w_dim - 1)
            pos = gy * 128 + gx
            hs = []
            if s == 0:
                lab = bx_v[rep, 4, :].astype(jnp.int32)
                c2 = lax.shift_right_logical(lab, 1)
                idxf_v[slot, 0, :] = b * (80 * 128) + pos
                idxp_v[slot, 0, :] = (b * 40 + c2) * (80 * 128) + pos
                box_off, obj_off = 32, 96
            else:
                if s == 1:
                    base_cls = b * (40 * 40 * 128) + (gy * 40 + gx) * 128
                else:
                    base_cls = (gy * 20 + gx) * (B * 128) + b * 128

                def build_cls(kk, carry, base_cls=base_cls, slot=slot):
                    for j in range(8):
                        idxc_v[slot, kk, pl.ds(j * 16, 16)] = (
                            base_cls + (kk * 8 + j))
                    return carry
                lax.fori_loop(0, 10, build_cls, 0)
                box_off, obj_off = ROWS_CLS, ROWS_CLS + 64
            for c4 in range(4):
                idxb_v[slot, 0, pl.ds(c4 * 16, 16)] = (
                    b * (4 * hw) + c4 * hw + pos)
            idxo_v[slot, 0, :] = b * hw + pos
            if s == 0:
                hs.append(pltpu.async_copy(
                    sumf3_r.at[idxf_v.at[slot, 0]],
                    vals_v.at[slot, pl.ds(0, 16)], sems[slot]))
                hs.append(pltpu.async_copy(
                    pk3_r.at[idxp_v.at[slot, 0]],
                    vals_v.at[slot, pl.ds(16, 16)], sems[slot]))
            else:
                for kk in range(10):
                    hs.append(pltpu.async_copy(
                        cls_tab.at[idxc_v.at[slot, kk]],
                        vals_v.at[slot, pl.ds(kk * 128, 128)], sems[slot]))
            hs.append(pltpu.async_copy(
                box_tab.at[idxb_v.at[slot, 0]],
                vals_v.at[slot, pl.ds(box_off, 64)], sems[slot]))
            hs.append(pltpu.async_copy(
                obj_tab.at[idxo_v.at[slot, 0]],
                vals_v.at[slot, pl.ds(obj_off, 16)], sems[slot]))
            handles[i] = hs
        for i in (4, 5):
            for cp in handles[i]:
                cp.wait()
            writeout(i)

    return k(sumf3, pk3, cls4, cls5, box3, box4, box5, obj3, obj4, obj5,
             boxes_sc)


def _sigmoid(x):
    return 1.0 / (1.0 + jnp.exp(-x))


def _l1pe(x):
    return jnp.log(1.0 + jnp.exp(-jnp.abs(x)))


def _softplus(x):
    return jnp.maximum(x, 0.0) + _l1pe(x)


def _atan(x):
    w1 = x / (1.0 + jnp.sqrt(1.0 + x * x))
    w = w1 / (1.0 + jnp.sqrt(1.0 + w1 * w1))
    w2 = w * w
    p = w * (1.0 + w2 * (-1.0 / 3.0 + w2 * (1.0 / 5.0 + w2 * (
        -1.0 / 7.0 + w2 * (1.0 / 9.0 + w2 * (-1.0 / 11.0 + w2 / 13.0))))))
    return 4.0 * p


def _tc_body(g_ref, o3_ref, o4_ref, o5_ref, btc_ref, lab_ref, img_ref,
             lo_ref, lbx_ref, lob_ref, lcl_ref):
    img = img_ref[0, 0]
    eps = 1e-7
    seg = (lax.broadcasted_iota(jnp.int32, (ROWS_CLS, 16), 0) % 16
           == lax.broadcasted_iota(jnp.int32, (ROWS_CLS, 16), 1)
           ).astype(jnp.float32)
    total_box = jnp.float32(0.0)
    total_obj = jnp.float32(0.0)
    total_cls = jnp.float32(0.0)
    total_pos = jnp.float32(0.0)
    o_refs = (o3_ref, o4_ref, o5_ref)
    for s, (w_dim, stride, mn, mx) in enumerate(SCALES):
        hw = w_dim * w_dim
        obj_sum = o_refs[s][:, 0, 0:1]
        box_b = jnp.zeros((B, 1), jnp.float32)
        cls_b = jnp.zeros((B, 1), jnp.float32)
        pos_b = jnp.zeros((B, 1), jnp.float32)
        nval_b = jnp.zeros((B, 1), jnp.float32)
        pos_l, mf_l, spo_l = [], [], []
        for ck in range(4):
            sl = slice(ck * 16, (ck + 1) * 16)
            cx = btc_ref[0, :, sl]
            cy = btc_ref[1, :, sl]
            tw = btc_ref[2, :, sl]
            th = btc_ref[3, :, sl]
            tmax = jnp.maximum(tw, th) * img
            maskf = ((tmax > mn) & (tmax <= mx)).astype(jnp.float32)
            tx1 = cx - 0.5 * tw
            ty1 = cy - 0.5 * th
            tx2 = cx + 0.5 * tw
            ty2 = cy + 0.5 * th
            gx = jnp.clip((cx * w_dim).astype(jnp.int32), 0, w_dim - 1)
            gy = jnp.clip((cy * w_dim).astype(jnp.int32), 0, w_dim - 1)
            yk = g_ref[s, ck]
            box_off, obj_off = (32, 96) if s == 0 else (ROWS_CLS,
                                                       ROWS_CLS + 64)
            r0 = yk[:, box_off:box_off + 16]
            r1 = yk[:, box_off + 16:box_off + 32]
            r2 = yk[:, box_off + 32:box_off + 48]
            r3 = yk[:, box_off + 48:box_off + 64]
            gobj = yk[:, obj_off:obj_off + 16]
            px = (gx.astype(jnp.float32) + _sigmoid(r0)) * stride / img
            py = (gy.astype(jnp.float32) + _sigmoid(r1)) * stride / img
            pw = jnp.exp(r2) * stride / img
            ph = jnp.exp(r3) * stride / img
            px1 = px - 0.5 * pw
            py1 = py - 0.5 * ph
            px2 = px + 0.5 * pw
            py2 = py + 0.5 * ph
            ap = (px2 - px1) * (py2 - py1)
            at = (tx2 - tx1) * (ty2 - ty1)
            iw = jnp.clip(jnp.minimum(px2, tx2) - jnp.maximum(px1, tx1),
                          0.0, None)
            ih = jnp.clip(jnp.minimum(py2, ty2) - jnp.maximum(py1, ty1),
                          0.0, None)
            inter = iw * ih
            union = ap + at - inter + eps
            iou = inter / union
            cw = jnp.maximum(px2, tx2) - jnp.minimum(px1, tx1)
            ch = jnp.maximum(py2, ty2) - jnp.minimum(py1, ty1)
            c2 = cw * cw + ch * ch + eps
            rho2 = ((px1 + px2 - tx1 - tx2) ** 2
                    + (py1 + py2 - ty1 - ty2) ** 2) / 4.0
            v = (4.0 / (jnp.pi ** 2)) * (
                _atan((tx2 - tx1) / (ty2 - ty1 + eps))
                - _atan((px2 - px1) / (py2 - py1 + eps))) ** 2
            alpha = v / (1.0 - iou + v + eps)
            ciou = iou - rho2 / c2 - alpha * v
            box_b = box_b + jnp.sum((1.0 - jnp.clip(ciou, -1.0, 1.0)) * maskf,
                                    axis=1, keepdims=True)
            if s == 0:
                sumf_t = yk[:, 0:16]
                packed = lax.bitcast_convert_type(yk[:, 16:32], jnp.uint32)
                labs = lab_ref[ck][:, 0:16]
                sel = jnp.where((labs & 1) == 1, packed >> 16,
                                packed & 0xFFFF)
                xlab = lax.bitcast_convert_type(
                    sel.astype(jnp.uint16), jnp.bfloat16).astype(jnp.float32)
                cls_t = sumf_t - xlab
            else:
                gcls = yk[:, 0:ROWS_CLS]
                labx = lab_ref[ck]
                cj = lax.broadcasted_iota(jnp.int32, (B, ROWS_CLS), 1) // 16
                oneh = (cj == labx).astype(jnp.float32)
                bce_cls = jnp.maximum(gcls, 0.0) - gcls * oneh + _l1pe(gcls)
                cls_t = jax.lax.dot(bce_cls, seg,
                                    precision=jax.lax.Precision.HIGHEST)
            cls_b = cls_b + jnp.sum(cls_t * maskf, axis=1, keepdims=True)
            pos_b = pos_b + jnp.sum(
                (jnp.maximum(gobj, 0.0) - gobj + _l1pe(gobj)) * maskf,
                axis=1, keepdims=True)
            nval_b = nval_b + jnp.sum(maskf, axis=1, keepdims=True)
            pos_l.append(gy * w_dim + gx)
            mf_l.append(maskf)
            spo_l.append(_softplus(gobj))
        neg_hit = jnp.zeros((B, 1), jnp.float32)
        for ck in range(4):
            dup = jnp.zeros((B, 16, 16), jnp.float32)
            for pk in range(ck + 1):
                eq = pos_l[pk][:, :, None] == pos_l[ck][:, None, :]
                if pk == ck:
                    tri = (lax.broadcasted_iota(jnp.int32, (B, 16, 16), 1)
                           < lax.broadcasted_iota(jnp.int32, (B, 16, 16), 2))
                    eq = eq & tri
                dup = dup + jnp.where(eq, mf_l[pk][:, :, None], 0.0)
            first = mf_l[ck] * (jnp.sum(dup, axis=1) == 0.0).astype(jnp.float32)
            neg_hit = neg_hit + jnp.sum(spo_l[ck] * first,
                                        axis=1, keepdims=True)
        neg_b = 0.05 * (obj_sum - neg_hit)
        fb_b = 0.1 * obj_sum / hw
        has = nval_b > 0.0
        total_obj = total_obj + jnp.sum(jnp.where(has, pos_b + neg_b, fb_b))
        total_box = total_box + jnp.sum(jnp.where(has, box_b, 0.0))
        total_cls = total_cls + jnp.sum(jnp.where(has, cls_b, 0.0))
        total_pos = total_pos + jnp.sum(jnp.where(has, nval_b, 0.0))
    norm = jnp.maximum(total_pos, 1.0)
    box_loss = total_box / norm
    cls_loss = total_cls / norm
    obj_loss = total_obj / (B * 3.0)
    loss = 7.5 * box_loss + 1.0 * obj_loss + 0.5 * cls_loss
    lo_ref[...] = jnp.reshape(loss, (1, 1))
    lbx_ref[...] = jnp.reshape(box_loss, (1, 1))
    lob_ref[...] = jnp.reshape(obj_loss, (1, 1))
    lcl_ref[...] = jnp.reshape(cls_loss, (1, 1))


def _tc_loss(g, o3, o4, o5, btc, lab_exp, img11):
    return pl.pallas_call(
        _tc_body,
        out_shape=[jax.ShapeDtypeStruct((1, 1), jnp.float32)] * 4,
    )(g, o3, o4, o5, btc, lab_exp, img11)


def kernel(p3_bbox, p3_obj, p3_cls, p4_bbox, p4_obj, p4_cls,
           p5_bbox, p5_obj, p5_cls, boxes, labels, img_size):
    img = jnp.asarray(img_size).astype(jnp.float32)
    boxes_p = jnp.pad(boxes.astype(jnp.float32),
                      ((0, 0), (0, NTP - NT), (0, 0)))
    labels_p = jnp.pad(labels.astype(jnp.int32), ((0, 0), (0, NTP - NT)))
    boxes_sc = jnp.concatenate([
        boxes_p.reshape(B, 4, 16, 4).transpose(0, 1, 3, 2).reshape(64, 4, 16),
        labels_p.astype(jnp.float32).reshape(64, 1, 16)], axis=1)
    boxes_tc = boxes_p.transpose(2, 0, 1)
    lin, osums = _relayout(p3_cls, p3_bbox, p3_obj,
                           p4_cls.transpose(0, 2, 3, 1), p4_bbox, p4_obj,
                           p5_cls.transpose(2, 3, 0, 1), p5_bbox, p5_obj)
    g = _sc_gather(*[t.reshape(-1) for t in lin], boxes_sc)
    lab_exp = jnp.broadcast_to(
        labels_p.reshape(B, 4, 1, 16).transpose(1, 0, 2, 3),
        (4, B, C, 16)).reshape(4, B, ROWS_CLS)
    loss, box_loss, obj_loss, cls_loss = _tc_loss(
        g, *osums, boxes_tc, lab_exp, img.reshape(1, 1))
    return (loss.reshape(()), box_loss.reshape(()),
            obj_loss.reshape(()), cls_loss.reshape(()))
